# Initial kernel scaffold; baseline (speedup 1.0000x reference)
import functools

import jax
import jax.numpy as jnp
from jax import lax
from jax.experimental import pallas as pl
from jax.experimental.pallas import tpu as pltpu
from jax.experimental.pallas import tpu_sc as plsc

CH = 128
HEADS = 8
KNN = 32
G = CH // HEADS
HID = 512
EPS = 1e-5

QKV_BLK = 512
TOPK_RB = 256
ATT_BLK = 128
GATHER_CHUNK = 128


def _lrelu(x):
    return jnp.where(x >= 0, x, 0.2 * x)


def _mm(a, w, bias):
    y = lax.dot_general(a, w, (((1,), (1,)), ((), ())),
                        preferred_element_type=jnp.float32)
    return y + bias


def _qkv_body(x_ref, wq_ref, bq_ref, wk_ref, bk_ref, wv_ref, bv_ref,
              q_ref, tab_ref):
    xb = x_ref[0]
    q_ref[0] = _lrelu(_mm(xb, wq_ref[...], bq_ref[...]))
    tab_ref[0] = _lrelu(_mm(xb, wk_ref[...], bk_ref[...]))
    tab_ref[1] = _lrelu(_mm(xb, wv_ref[...], bv_ref[...]))
    tab_ref[2] = xb


def _qkv(x, wq, bq2, wk, bk2, wv, bv2):
    b, n, _ = x.shape
    nb = n // QKV_BLK
    wspec = pl.BlockSpec((CH, CH), lambda i, j: (0, 0))
    bspec = pl.BlockSpec((1, CH), lambda i, j: (0, 0))
    return pl.pallas_call(
        _qkv_body,
        grid=(b, nb),
        in_specs=[
            pl.BlockSpec((1, QKV_BLK, CH), lambda i, j: (i, j, 0)),
            wspec, bspec, wspec, bspec, wspec, bspec,
        ],
        out_specs=[
            pl.BlockSpec((1, QKV_BLK, CH), lambda i, j: (i, j, 0)),
            pl.BlockSpec((3, QKV_BLK, CH), lambda i, j: (0, i * nb + j, 0)),
        ],
        out_shape=[
            jax.ShapeDtypeStruct((b, n, CH), jnp.float32),
            jax.ShapeDtypeStruct((3, b * n, CH), jnp.float32),
        ],
    )(x, wq, bq2, wk, bk2, wv, bv2)


def _topk_body(n, fb_ref, fa_ref, idx_ref):
    fb = fb_ref[...]
    fa = fa_ref[...]
    bidx = pl.program_id(0)
    s = lax.dot_general(fb, fa, (((1,), (1,)), ((), ())),
                        preferred_element_type=jnp.float32)
    xxb = jnp.sum(fb * fb, axis=1, keepdims=True)
    ones8 = jnp.ones((8, CH), jnp.float32)
    xxa8 = lax.dot_general(ones8, fa * fa, (((1,), (1,)), ((), ())),
                           preferred_element_type=jnp.float32)
    pd = 2.0 * s - xxb - xxa8[0:1, :]
    iota = lax.broadcasted_iota(jnp.int32, (TOPK_RB, n), 1)
    neg = jnp.float32(-3.0e38)
    cols = []
    vals = pd
    for _ in range(KNN):
        m = jnp.max(vals, axis=1, keepdims=True)
        cand = jnp.where(vals >= m, iota, n)
        sel = jnp.min(cand, axis=1, keepdims=True)
        cols.append(sel)
        vals = jnp.where(iota == sel, neg, vals)
    idx = jnp.concatenate(cols, axis=1)
    idx_ref[0] = idx + bidx * n


def _topk(table2d, t, b, n):
    nb = n // TOPK_RB
    return pl.pallas_call(
        functools.partial(_topk_body, n),
        grid=(b, nb),
        in_specs=[
            pl.BlockSpec((TOPK_RB, CH), lambda i, j: ((t * b + i) * nb + j, 0)),
            pl.BlockSpec((n, CH), lambda i, j: (t * b + i, 0)),
        ],
        out_specs=pl.BlockSpec((1, TOPK_RB, KNN), lambda i, j: (i, j, 0)),
        out_shape=jax.ShapeDtypeStruct((b, n, KNN), jnp.int32),
    )(table2d, table2d)


def _sc_gather(table2d, cidx):
    info = plsc.get_sparse_core_info()
    nw = info.num_cores * info.num_subcores
    rows = cidx.shape[0]
    rw = rows // nw
    c = GATHER_CHUNK
    steps = rw // c
    mesh = plsc.VectorSubcoreMesh(core_axis_name="c", subcore_axis_name="s")

    @functools.partial(
        pl.kernel,
        out_type=jax.ShapeDtypeStruct((rows, CH), jnp.float32),
        mesh=mesh,
        scratch_types=[
            pltpu.VMEM((rw,), jnp.int32),
            pltpu.VMEM((2, c, CH), jnp.float32),
            pltpu.SemaphoreType.DMA,
            pltpu.SemaphoreType.DMA,
        ],
    )
    def gather_kernel(tab_hbm, idx_hbm, out_hbm, idx_v, rows_v, gsem, osem):
        wid = lax.axis_index("s") * info.num_cores + lax.axis_index("c")
        base = wid * rw
        pltpu.sync_copy(idx_hbm.at[pl.ds(base, rw)], idx_v)

        def start(i, slot):
            pltpu.async_copy(tab_hbm.at[idx_v.at[pl.ds(i * c, c)]],
                             rows_v.at[slot], gsem)

        def wait_gather(i, slot):
            pltpu.make_async_copy(tab_hbm.at[idx_v.at[pl.ds(i * c, c)]],
                                  rows_v.at[slot], gsem).wait()

        def start_out(i, slot):
            pltpu.async_copy(rows_v.at[slot],
                             out_hbm.at[pl.ds(base + i * c, c)], osem)

        def wait_out(i, slot):
            pltpu.make_async_copy(rows_v.at[slot],
                                  out_hbm.at[pl.ds(base + i * c, c)],
                                  osem).wait()

        start(0, 0)

        def body(i, _):
            slot = lax.rem(i, 2)
            nslot = lax.rem(i + 1, 2)
            start(i + 1, nslot)
            wait_gather(i, slot)
            start_out(i, slot)
            wait_out(i, slot)
            return 0

        lax.fori_loop(0, steps - 1, body, 0)
        last = steps - 1
        lslot = lax.rem(last, 2)
        wait_gather(last, lslot)
        start_out(last, lslot)
        wait_out(last, lslot)

    return gather_kernel(table2d, cidx)


def _attn_body(gk_ref, gv_ref, gp_ref, q_ref, x_ref,
               wpb_ref, bpb_ref, we1_ref, be1_ref, we2_ref, be2_ref,
               wm1_ref, bm1_ref, wm2_ref, bm2_ref, o_ref):
    qb = q_ref[0]
    xb = x_ref[0]

    def rep(a):
        return jnp.broadcast_to(a[:, None, :], (ATT_BLK, KNN, CH)).reshape(
            ATT_BLK * KNN, CH)

    mpos = gp_ref[...] - rep(xb)
    posb = _lrelu(_mm(mpos, wpb_ref[...], bpb_ref[...]))
    vec = gk_ref[...] - rep(qb) + posb
    val = gv_ref[...] + posb
    e = _lrelu(_mm(vec, we1_ref[...], be1_ref[...]))
    e = _lrelu(_mm(e, we2_ref[...], be2_ref[...]))
    e3 = e.reshape(ATT_BLK, KNN, G)
    m = jnp.max(e3, axis=1, keepdims=True)
    p = jnp.exp(e3 - m)
    w = p / jnp.sum(p, axis=1, keepdims=True)
    ci = lax.broadcasted_iota(jnp.int32, (G, CH), 1) // HEADS
    gi = lax.broadcasted_iota(jnp.int32, (G, CH), 0)
    expand = (ci == gi).astype(jnp.float32)
    wfull = lax.dot_general(w.reshape(ATT_BLK * KNN, G), expand,
                            (((1,), (0,)), ((), ())),
                            preferred_element_type=jnp.float32)
    attn = jnp.sum((wfull * val).reshape(ATT_BLK, KNN, CH), axis=1)
    y = xb + attn
    h = jax.nn.silu(_mm(y, wm1_ref[...], bm1_ref[...]))
    h = jax.nn.silu(_mm(h, wm2_ref[...], bm2_ref[...]))
    o_ref[0] = y + h


def _attn(gath, q, x, wpb, bpb2, we1, be12, we2, be22, wm1, bm12, wm2, bm22):
    b, n, _ = x.shape
    nb = n // ATT_BLK

    def gspec(t):
        return pl.BlockSpec((ATT_BLK * KNN, CH),
                            lambda i, j, t=t: (t * b * nb + i * nb + j, 0))

    def cspec(shape):
        return pl.BlockSpec(shape, lambda i, j: (0, 0))

    return pl.pallas_call(
        _attn_body,
        grid=(b, nb),
        in_specs=[
            gspec(0), gspec(1), gspec(2),
            pl.BlockSpec((1, ATT_BLK, CH), lambda i, j: (i, j, 0)),
            pl.BlockSpec((1, ATT_BLK, CH), lambda i, j: (i, j, 0)),
            cspec((CH, CH)), cspec((1, CH)),
            cspec((G, CH)), cspec((1, G)),
            cspec((G, G)), cspec((1, G)),
            cspec((HID, CH)), cspec((1, HID)),
            cspec((CH, HID)), cspec((1, CH)),
        ],
        out_specs=pl.BlockSpec((1, ATT_BLK, CH), lambda i, j: (i, j, 0)),
        out_shape=jax.ShapeDtypeStruct((b, n, CH), jnp.float32),
    )(gath, gath, gath, q, x, wpb, bpb2, we1, be12, we2, be22,
      wm1, bm12, wm2, bm22)


def kernel(x, Wq, gq, bq, Wk, gk, bk, Wv, gv, bv, Wpb, gpb, bpb,
           We1, ge1, be1, We2, ge2, be2, Wm1, gm1, bm1, Wm2, gm2, bm2):
    b, n, _ = x.shape
    bn = b * n
    scale = 1.0 / jnp.sqrt(jnp.float32(1.0 + EPS))

    def fold(w, g):
        return w * (g * scale)[:, None]

    q, table = _qkv(x,
                    fold(Wq, gq), bq[None, :],
                    fold(Wk, gk), bk[None, :],
                    fold(Wv, gv), bv[None, :])
    table2d = table.reshape(3 * bn, CH)

    gk_idx = _topk(table2d, 0, b, n)
    gv_idx = _topk(table2d, 1, b, n)

    cidx = jnp.concatenate([
        gk_idx.reshape(-1),
        gv_idx.reshape(-1) + bn,
        gk_idx.reshape(-1) + 2 * bn,
    ])
    gath = _sc_gather(table2d, cidx)

    return _attn(gath, q, x,
                 fold(Wpb, gpb), bpb[None, :],
                 fold(We1, ge1), be1[None, :],
                 fold(We2, ge2), be2[None, :],
                 fold(Wm1, gm1), bm1[None, :],
                 fold(Wm2, gm2), bm2[None, :])


# speedup vs baseline: 5.1067x; 5.1067x over previous
import functools

import jax
import jax.numpy as jnp
from jax import lax
from jax.experimental import pallas as pl
from jax.experimental.pallas import tpu as pltpu
from jax.experimental.pallas import tpu_sc as plsc

CH = 128
HEADS = 8
KNN = 32
G = CH // HEADS
HID = 512
EPS = 1e-5

QKV_BLK = 512
TOPK_RB = 256
ATT_BLK = 128
GATHER_CHUNK = 128


def _lrelu(x):
    return jnp.where(x >= 0, x, 0.2 * x)


def _dot_bf(a, w):
    return lax.dot_general(a.astype(jnp.bfloat16), w.astype(jnp.bfloat16),
                           (((1,), (1,)), ((), ())),
                           preferred_element_type=jnp.float32)


def _mmsb(a, w, s, bias):
    return _dot_bf(a, w) * s + bias


def _qkv_body(x_ref, wq_ref, sq_ref, bq_ref, wk_ref, sk_ref, bk_ref,
              wv_ref, sv_ref, bv_ref, q_ref, tab_ref):
    xb = x_ref[0]
    q_ref[0] = _lrelu(_mmsb(xb, wq_ref[...], sq_ref[...], bq_ref[...]))
    tab_ref[0] = _lrelu(_mmsb(xb, wk_ref[...], sk_ref[...], bk_ref[...]))
    tab_ref[1] = _lrelu(_mmsb(xb, wv_ref[...], sv_ref[...], bv_ref[...]))
    tab_ref[2] = xb


def _qkv(x, wq, sq2, bq2, wk, sk2, bk2, wv, sv2, bv2):
    b, n, _ = x.shape
    nb = n // QKV_BLK
    wspec = pl.BlockSpec((CH, CH), lambda i, j: (0, 0))
    bspec = pl.BlockSpec((1, CH), lambda i, j: (0, 0))
    return pl.pallas_call(
        _qkv_body,
        grid=(b, nb),
        in_specs=[
            pl.BlockSpec((1, QKV_BLK, CH), lambda i, j: (i, j, 0)),
            wspec, bspec, bspec, wspec, bspec, bspec, wspec, bspec, bspec,
        ],
        out_specs=[
            pl.BlockSpec((1, QKV_BLK, CH), lambda i, j: (i, j, 0)),
            pl.BlockSpec((3, QKV_BLK, CH), lambda i, j: (0, i * nb + j, 0)),
        ],
        out_shape=[
            jax.ShapeDtypeStruct((b, n, CH), jnp.float32),
            jax.ShapeDtypeStruct((3, b * n, CH), jnp.float32),
        ],
    )(x, wq, sq2, bq2, wk, sk2, bk2, wv, sv2, bv2)


def _topk_body(n, fb_ref, fa_ref, idx_ref):
    fb = fb_ref[...]
    fa = fa_ref[...]
    bidx = pl.program_id(0)
    s = _dot_bf(fb, fa)
    xxb = jnp.sum(fb * fb, axis=1, keepdims=True)
    ones8 = jnp.ones((8, CH), jnp.float32)
    xxa8 = lax.dot_general(ones8, fa * fa, (((1,), (1,)), ((), ())),
                           preferred_element_type=jnp.float32,
                           precision=lax.Precision.HIGHEST)
    pd = ((-xxb) - (jnp.float32(-2.0) * s)) - xxa8[0:1, :]
    iota = lax.broadcasted_iota(jnp.int32, (TOPK_RB, n), 1)
    neg = jnp.float32(-3.0e38)
    cols = []
    vals = pd
    for _ in range(KNN):
        m = jnp.max(vals, axis=1, keepdims=True)
        cand = jnp.where(vals >= m, iota, n)
        sel = jnp.min(cand, axis=1, keepdims=True)
        cols.append(sel)
        vals = jnp.where(iota == sel, neg, vals)
    idx = jnp.concatenate(cols, axis=1)
    idx_ref[0] = idx + bidx * n


def _topk(table2d, t, b, n):
    nb = n // TOPK_RB
    return pl.pallas_call(
        functools.partial(_topk_body, n),
        grid=(b, nb),
        in_specs=[
            pl.BlockSpec((TOPK_RB, CH), lambda i, j: ((t * b + i) * nb + j, 0)),
            pl.BlockSpec((n, CH), lambda i, j: (t * b + i, 0)),
        ],
        out_specs=pl.BlockSpec((1, TOPK_RB, KNN), lambda i, j: (i, j, 0)),
        out_shape=jax.ShapeDtypeStruct((b, n, KNN), jnp.int32),
    )(table2d, table2d)


def _sc_gather(table2d, cidx):
    info = plsc.get_sparse_core_info()
    nw = info.num_cores * info.num_subcores
    rows = cidx.shape[0]
    rw = rows // nw
    c = GATHER_CHUNK
    steps = rw // c
    mesh = plsc.VectorSubcoreMesh(core_axis_name="c", subcore_axis_name="s")

    @functools.partial(
        pl.kernel,
        out_type=jax.ShapeDtypeStruct((rows, CH), jnp.float32),
        mesh=mesh,
        scratch_types=[
            pltpu.VMEM((rw,), jnp.int32),
            pltpu.VMEM((2, c, CH), jnp.float32),
            pltpu.SemaphoreType.DMA,
            pltpu.SemaphoreType.DMA,
        ],
    )
    def gather_kernel(tab_hbm, idx_hbm, out_hbm, idx_v, rows_v, gsem, osem):
        wid = lax.axis_index("s") * info.num_cores + lax.axis_index("c")
        base = wid * rw
        pltpu.sync_copy(idx_hbm.at[pl.ds(base, rw)], idx_v)

        def start(i, slot):
            pltpu.async_copy(tab_hbm.at[idx_v.at[pl.ds(i * c, c)]],
                             rows_v.at[slot], gsem)

        def wait_gather(i, slot):
            pltpu.make_async_copy(tab_hbm.at[idx_v.at[pl.ds(i * c, c)]],
                                  rows_v.at[slot], gsem).wait()

        def start_out(i, slot):
            pltpu.async_copy(rows_v.at[slot],
                             out_hbm.at[pl.ds(base + i * c, c)], osem)

        def wait_out(i, slot):
            pltpu.make_async_copy(rows_v.at[slot],
                                  out_hbm.at[pl.ds(base + i * c, c)],
                                  osem).wait()

        start(0, 0)

        def body(i, _):
            slot = lax.rem(i, 2)
            nslot = lax.rem(i + 1, 2)
            start(i + 1, nslot)
            wait_gather(i, slot)
            start_out(i, slot)
            wait_out(i, slot)
            return 0

        lax.fori_loop(0, steps - 1, body, 0)
        last = steps - 1
        lslot = lax.rem(last, 2)
        wait_gather(last, lslot)
        start_out(last, lslot)
        wait_out(last, lslot)

    return gather_kernel(table2d, cidx)


def _attn_body(gk_ref, gv_ref, gp_ref, q_ref, x_ref,
               wpb_ref, spb_ref, bpb_ref, we1_ref, se1_ref, be1_ref,
               we2_ref, se2_ref, be2_ref, wm1_ref, sm1_ref, bm1_ref,
               wm2_ref, sm2_ref, bm2_ref, o_ref):
    qb = q_ref[0]
    xb = x_ref[0]

    def rep(a):
        return jnp.broadcast_to(a[:, None, :], (ATT_BLK, KNN, CH)).reshape(
            ATT_BLK * KNN, CH)

    mpos = gp_ref[...] - rep(xb)
    posb = _lrelu(_mmsb(mpos, wpb_ref[...], spb_ref[...], bpb_ref[...]))
    vec = gk_ref[...] - rep(qb) + posb
    val = gv_ref[...] + posb
    e = _lrelu(_mmsb(vec, we1_ref[...], se1_ref[...], be1_ref[...]))
    e = _lrelu(_mmsb(e, we2_ref[...], se2_ref[...], be2_ref[...]))
    e3 = e.reshape(ATT_BLK, KNN, G)
    m = jnp.max(e3, axis=1, keepdims=True)
    p = jnp.exp(e3 - m)
    w = p / jnp.sum(p, axis=1, keepdims=True)
    ci = lax.broadcasted_iota(jnp.int32, (CH, G), 0) // HEADS
    gi = lax.broadcasted_iota(jnp.int32, (CH, G), 1)
    expand_t = (ci == gi).astype(jnp.float32)
    wfull = _dot_bf(w.reshape(ATT_BLK * KNN, G), expand_t)
    valb = val.astype(jnp.bfloat16).astype(jnp.float32)
    attn = jnp.sum((wfull * valb).reshape(ATT_BLK, KNN, CH), axis=1)
    y = xb + attn
    h = jax.nn.silu(_mmsb(y, wm1_ref[...], sm1_ref[...], bm1_ref[...]))
    h = jax.nn.silu(_mmsb(h, wm2_ref[...], sm2_ref[...], bm2_ref[...]))
    o_ref[0] = y + h


def _attn(gath, q, x, wpb, spb2, bpb2, we1, se12, be12, we2, se22, be22,
          wm1, sm12, bm12, wm2, sm22, bm22):
    b, n, _ = x.shape
    nb = n // ATT_BLK

    def gspec(t):
        return pl.BlockSpec((ATT_BLK * KNN, CH),
                            lambda i, j, t=t: (t * b * nb + i * nb + j, 0))

    def cspec(shape):
        return pl.BlockSpec(shape, lambda i, j: (0, 0))

    return pl.pallas_call(
        _attn_body,
        grid=(b, nb),
        in_specs=[
            gspec(0), gspec(1), gspec(2),
            pl.BlockSpec((1, ATT_BLK, CH), lambda i, j: (i, j, 0)),
            pl.BlockSpec((1, ATT_BLK, CH), lambda i, j: (i, j, 0)),
            cspec((CH, CH)), cspec((1, CH)), cspec((1, CH)),
            cspec((G, CH)), cspec((1, G)), cspec((1, G)),
            cspec((G, G)), cspec((1, G)), cspec((1, G)),
            cspec((HID, CH)), cspec((1, HID)), cspec((1, HID)),
            cspec((CH, HID)), cspec((1, CH)), cspec((1, CH)),
        ],
        out_specs=pl.BlockSpec((1, ATT_BLK, CH), lambda i, j: (i, j, 0)),
        out_shape=jax.ShapeDtypeStruct((b, n, CH), jnp.float32),
    )(gath, gath, gath, q, x, wpb, spb2, bpb2, we1, se12, be12,
      we2, se22, be22, wm1, sm12, bm12, wm2, sm22, bm22)


def kernel(x, Wq, gq, bq, Wk, gk, bk, Wv, gv, bv, Wpb, gpb, bpb,
           We1, ge1, be1, We2, ge2, be2, Wm1, gm1, bm1, Wm2, gm2, bm2):
    b, n, _ = x.shape
    bn = b * n
    scale = 1.0 / jnp.sqrt(jnp.float32(1.0 + EPS))

    def sv(g):
        return (g * scale)[None, :]

    q, table = _qkv(x,
                    Wq, sv(gq), bq[None, :],
                    Wk, sv(gk), bk[None, :],
                    Wv, sv(gv), bv[None, :])
    table2d = table.reshape(3 * bn, CH)

    gk_idx = _topk(table2d, 0, b, n)
    gv_idx = _topk(table2d, 1, b, n)

    cidx = jnp.concatenate([
        gk_idx.reshape(-1),
        gv_idx.reshape(-1) + bn,
        gk_idx.reshape(-1) + 2 * bn,
    ])
    gath = _sc_gather(table2d, cidx)

    return _attn(gath, q, x,
                 Wpb, sv(gpb), bpb[None, :],
                 We1, sv(ge1), be1[None, :],
                 We2, sv(ge2), be2[None, :],
                 Wm1, sv(gm1), bm1[None, :],
                 Wm2, sv(gm2), bm2[None, :])


# speedup vs baseline: 6.2669x; 1.2272x over previous
import functools

import jax
import jax.numpy as jnp
from jax import lax
from jax.experimental import pallas as pl
from jax.experimental.pallas import tpu as pltpu
from jax.experimental.pallas import tpu_sc as plsc

CH = 128
HEADS = 8
KNN = 32
G = CH // HEADS
HID = 512
EPS = 1e-5

QKV_BLK = 512
TOPK_RB = 512
ATT_BLK = 128
GATHER_CHUNK = 128


def _lrelu(x):
    return jnp.where(x >= 0, x, 0.2 * x)


def _dot_bf(a, w):
    return lax.dot_general(a.astype(jnp.bfloat16), w.astype(jnp.bfloat16),
                           (((1,), (1,)), ((), ())),
                           preferred_element_type=jnp.float32)


def _mmsb(a, w, s, bias):
    return _dot_bf(a, w) * s + bias


def _qkv_body(x_ref, wq_ref, sq_ref, bq_ref, wk_ref, sk_ref, bk_ref,
              wv_ref, sv_ref, bv_ref, q_ref, tab_ref):
    xb = x_ref[0]
    q_ref[0] = _lrelu(_mmsb(xb, wq_ref[...], sq_ref[...], bq_ref[...]))
    tab_ref[0] = _lrelu(_mmsb(xb, wk_ref[...], sk_ref[...], bk_ref[...]))
    tab_ref[1] = _lrelu(_mmsb(xb, wv_ref[...], sv_ref[...], bv_ref[...]))
    tab_ref[2] = xb


def _qkv(x, wq, sq2, bq2, wk, sk2, bk2, wv, sv2, bv2):
    b, n, _ = x.shape
    nb = n // QKV_BLK
    wspec = pl.BlockSpec((CH, CH), lambda i, j: (0, 0))
    bspec = pl.BlockSpec((1, CH), lambda i, j: (0, 0))
    return pl.pallas_call(
        _qkv_body,
        grid=(b, nb),
        in_specs=[
            pl.BlockSpec((1, QKV_BLK, CH), lambda i, j: (i, j, 0)),
            wspec, bspec, bspec, wspec, bspec, bspec, wspec, bspec, bspec,
        ],
        out_specs=[
            pl.BlockSpec((1, QKV_BLK, CH), lambda i, j: (i, j, 0)),
            pl.BlockSpec((3, QKV_BLK, CH), lambda i, j: (0, i * nb + j, 0)),
        ],
        out_shape=[
            jax.ShapeDtypeStruct((b, n, CH), jnp.float32),
            jax.ShapeDtypeStruct((3, b * n, CH), jnp.float32),
        ],
    )(x, wq, sq2, bq2, wk, sk2, bk2, wv, sv2, bv2)


def _topk_body(n, fb_ref, fa_ref, idx_ref):
    fb = fb_ref[...]
    fa = fa_ref[...]
    bidx = pl.program_id(0)
    s = _dot_bf(fb, fa)
    xxb = jnp.sum(fb * fb, axis=1, keepdims=True)
    ones8 = jnp.ones((8, CH), jnp.float32)
    xxa8 = lax.dot_general(ones8, fa * fa, (((1,), (1,)), ((), ())),
                           preferred_element_type=jnp.float32,
                           precision=lax.Precision.HIGHEST)
    pd = ((-xxb) - (jnp.float32(-2.0) * s)) - xxa8[0:1, :]
    iota = lax.broadcasted_iota(jnp.int32, (TOPK_RB, n), 1)
    neg = jnp.float32(-3.0e38)
    cols = []
    vals = pd
    for _ in range(KNN):
        m = jnp.max(vals, axis=1, keepdims=True)
        cand = jnp.where(vals >= m, iota, n)
        sel = jnp.min(cand, axis=1, keepdims=True)
        cols.append(sel)
        vals = jnp.where(iota == sel, neg, vals)
    idx = jnp.concatenate(cols, axis=1)
    idx_ref[0] = idx + bidx * n


def _topk(table2d, t, b, n):
    nb = n // TOPK_RB
    return pl.pallas_call(
        functools.partial(_topk_body, n),
        grid=(b, nb),
        in_specs=[
            pl.BlockSpec((TOPK_RB, CH), lambda i, j: ((t * b + i) * nb + j, 0)),
            pl.BlockSpec((n, CH), lambda i, j: (t * b + i, 0)),
        ],
        out_specs=pl.BlockSpec((1, TOPK_RB, KNN), lambda i, j: (i, j, 0)),
        out_shape=jax.ShapeDtypeStruct((b, n, KNN), jnp.int32),
    )(table2d, table2d)


def _sc_gather(table2d, cidx):
    info = plsc.get_sparse_core_info()
    nw = info.num_cores * info.num_subcores
    rows = cidx.shape[0]
    rw = rows // nw
    c = GATHER_CHUNK
    steps = rw // c
    mesh = plsc.VectorSubcoreMesh(core_axis_name="c", subcore_axis_name="s")

    @functools.partial(
        pl.kernel,
        out_type=jax.ShapeDtypeStruct((rows, CH), jnp.float32),
        mesh=mesh,
        scratch_types=[
            pltpu.VMEM((rw,), jnp.int32),
            pltpu.VMEM((2, c, CH), jnp.float32),
            pltpu.SemaphoreType.DMA,
            pltpu.SemaphoreType.DMA,
        ],
    )
    def gather_kernel(tab_hbm, idx_hbm, out_hbm, idx_v, rows_v, gsem, osem):
        wid = lax.axis_index("s") * info.num_cores + lax.axis_index("c")
        base = wid * rw
        pltpu.sync_copy(idx_hbm.at[pl.ds(base, rw)], idx_v)

        def start(i, slot):
            pltpu.async_copy(tab_hbm.at[idx_v.at[pl.ds(i * c, c)]],
                             rows_v.at[slot], gsem)

        def wait_gather(i, slot):
            pltpu.make_async_copy(tab_hbm.at[idx_v.at[pl.ds(i * c, c)]],
                                  rows_v.at[slot], gsem).wait()

        def start_out(i, slot):
            pltpu.async_copy(rows_v.at[slot],
                             out_hbm.at[pl.ds(base + i * c, c)], osem)

        def wait_out(i, slot):
            pltpu.make_async_copy(rows_v.at[slot],
                                  out_hbm.at[pl.ds(base + i * c, c)],
                                  osem).wait()

        start(0, 0)

        def body(i, _):
            slot = lax.rem(i, 2)
            nslot = lax.rem(i + 1, 2)
            start(i + 1, nslot)
            wait_gather(i, slot)
            start_out(i, slot)
            wait_out(i, slot)
            return 0

        lax.fori_loop(0, steps - 1, body, 0)
        last = steps - 1
        lslot = lax.rem(last, 2)
        wait_gather(last, lslot)
        start_out(last, lslot)
        wait_out(last, lslot)

    return gather_kernel(table2d, cidx)


def _attn_body(gk_ref, gv_ref, gp_ref, q_ref, x_ref,
               wpb_ref, spb_ref, bpb_ref, we1_ref, se1_ref, be1_ref,
               we2_ref, se2_ref, be2_ref, wm1_ref, sm1_ref, bm1_ref,
               wm2_ref, sm2_ref, bm2_ref, o_ref):
    qb = q_ref[0]
    xb = x_ref[0]

    def rep(a):
        return jnp.broadcast_to(a[:, None, :], (ATT_BLK, KNN, CH)).reshape(
            ATT_BLK * KNN, CH)

    mpos = gp_ref[...] - rep(xb)
    posb = _lrelu(_mmsb(mpos, wpb_ref[...], spb_ref[...], bpb_ref[...]))
    vec = gk_ref[...] - rep(qb) + posb
    val = gv_ref[...] + posb
    e = _lrelu(_mmsb(vec, we1_ref[...], se1_ref[...], be1_ref[...]))
    e = _lrelu(_mmsb(e, we2_ref[...], se2_ref[...], be2_ref[...]))
    e3 = e.reshape(ATT_BLK, KNN, G)
    m = jnp.max(e3, axis=1, keepdims=True)
    p = jnp.exp(e3 - m)
    w = p / jnp.sum(p, axis=1, keepdims=True)
    ci = lax.broadcasted_iota(jnp.int32, (CH, G), 0) // HEADS
    gi = lax.broadcasted_iota(jnp.int32, (CH, G), 1)
    expand_t = (ci == gi).astype(jnp.float32)
    wfull = _dot_bf(w.reshape(ATT_BLK * KNN, G), expand_t)
    valb = val.astype(jnp.bfloat16).astype(jnp.float32)
    attn = jnp.sum((wfull * valb).reshape(ATT_BLK, KNN, CH), axis=1)
    y = xb + attn
    h = jax.nn.silu(_mmsb(y, wm1_ref[...], sm1_ref[...], bm1_ref[...]))
    h = jax.nn.silu(_mmsb(h, wm2_ref[...], sm2_ref[...], bm2_ref[...]))
    o_ref[0] = y + h


def _attn(gath_kp, gath_v, q, x, wpb, spb2, bpb2, we1, se12, be12,
          we2, se22, be22, wm1, sm12, bm12, wm2, sm22, bm22):
    b, n, _ = x.shape
    nb = n // ATT_BLK

    def gspec(t):
        return pl.BlockSpec((ATT_BLK * KNN, CH),
                            lambda i, j, t=t: (t * b * nb + i * nb + j, 0))

    def cspec(shape):
        return pl.BlockSpec(shape, lambda i, j: (0, 0))

    return pl.pallas_call(
        _attn_body,
        grid=(b, nb),
        in_specs=[
            gspec(0), gspec(0), gspec(1),
            pl.BlockSpec((1, ATT_BLK, CH), lambda i, j: (i, j, 0)),
            pl.BlockSpec((1, ATT_BLK, CH), lambda i, j: (i, j, 0)),
            cspec((CH, CH)), cspec((1, CH)), cspec((1, CH)),
            cspec((G, CH)), cspec((1, G)), cspec((1, G)),
            cspec((G, G)), cspec((1, G)), cspec((1, G)),
            cspec((HID, CH)), cspec((1, HID)), cspec((1, HID)),
            cspec((CH, HID)), cspec((1, CH)), cspec((1, CH)),
        ],
        out_specs=pl.BlockSpec((1, ATT_BLK, CH), lambda i, j: (i, j, 0)),
        out_shape=jax.ShapeDtypeStruct((b, n, CH), jnp.float32),
    )(gath_kp, gath_v, gath_kp, q, x, wpb, spb2, bpb2, we1, se12, be12,
      we2, se22, be22, wm1, sm12, bm12, wm2, sm22, bm22)


def kernel(x, Wq, gq, bq, Wk, gk, bk, Wv, gv, bv, Wpb, gpb, bpb,
           We1, ge1, be1, We2, ge2, be2, Wm1, gm1, bm1, Wm2, gm2, bm2):
    b, n, _ = x.shape
    bn = b * n
    scale = 1.0 / jnp.sqrt(jnp.float32(1.0 + EPS))

    def sv(g):
        return (g * scale)[None, :]

    q, table = _qkv(x,
                    Wq, sv(gq), bq[None, :],
                    Wk, sv(gk), bk[None, :],
                    Wv, sv(gv), bv[None, :])
    table2d = table.reshape(3 * bn, CH)

    gk_idx = _topk(table2d, 0, b, n)
    cidx_kp = jnp.concatenate([
        gk_idx.reshape(-1),
        gk_idx.reshape(-1) + 2 * bn,
    ])
    gath_kp = _sc_gather(table2d, cidx_kp)

    gv_idx = _topk(table2d, 1, b, n)
    gath_v = _sc_gather(table2d, gv_idx.reshape(-1) + bn)

    return _attn(gath_kp, gath_v, q, x,
                 Wpb, sv(gpb), bpb[None, :],
                 We1, sv(ge1), be1[None, :],
                 We2, sv(ge2), be2[None, :],
                 Wm1, sv(gm1), bm1[None, :],
                 Wm2, sv(gm2), bm2[None, :])


# speedup vs baseline: 6.6467x; 1.0606x over previous
import functools

import jax
import jax.numpy as jnp
from jax import lax
from jax.experimental import pallas as pl
from jax.experimental.pallas import tpu as pltpu
from jax.experimental.pallas import tpu_sc as plsc

CH = 128
HEADS = 8
KNN = 32
G = CH // HEADS
HID = 512
EPS = 1e-5

QKV_BLK = 512
TOPK_RB = 256
TOPK_CHAINS = 4
ATT_BLK = 128
GATHER_CHUNK = 128


def _lrelu(x):
    return jnp.where(x >= 0, x, 0.2 * x)


def _dot_bf(a, w):
    return lax.dot_general(a.astype(jnp.bfloat16), w.astype(jnp.bfloat16),
                           (((1,), (1,)), ((), ())),
                           preferred_element_type=jnp.float32)


def _mmsb(a, w, s, bias):
    return _dot_bf(a, w) * s + bias


def _qkv_body(x_ref, wq_ref, sq_ref, bq_ref, wk_ref, sk_ref, bk_ref,
              wv_ref, sv_ref, bv_ref, q_ref, tab_ref):
    xb = x_ref[0]
    q_ref[0] = _lrelu(_mmsb(xb, wq_ref[...], sq_ref[...], bq_ref[...]))
    tab_ref[0] = _lrelu(_mmsb(xb, wk_ref[...], sk_ref[...], bk_ref[...]))
    tab_ref[1] = _lrelu(_mmsb(xb, wv_ref[...], sv_ref[...], bv_ref[...]))
    tab_ref[2] = xb


def _qkv(x, wq, sq2, bq2, wk, sk2, bk2, wv, sv2, bv2):
    b, n, _ = x.shape
    nb = n // QKV_BLK
    wspec = pl.BlockSpec((CH, CH), lambda i, j: (0, 0))
    bspec = pl.BlockSpec((1, CH), lambda i, j: (0, 0))
    return pl.pallas_call(
        _qkv_body,
        grid=(b, nb),
        in_specs=[
            pl.BlockSpec((1, QKV_BLK, CH), lambda i, j: (i, j, 0)),
            wspec, bspec, bspec, wspec, bspec, bspec, wspec, bspec, bspec,
        ],
        out_specs=[
            pl.BlockSpec((1, QKV_BLK, CH), lambda i, j: (i, j, 0)),
            pl.BlockSpec((3, QKV_BLK, CH), lambda i, j: (0, i * nb + j, 0)),
        ],
        out_shape=[
            jax.ShapeDtypeStruct((b, n, CH), jnp.float32),
            jax.ShapeDtypeStruct((3, b * n, CH), jnp.float32),
        ],
    )(x, wq, sq2, bq2, wk, sk2, bk2, wv, sv2, bv2)


def _topk_body(n, fb_ref, fa_ref, idx_ref):
    blk = TOPK_RB * TOPK_CHAINS
    fb = fb_ref[...]
    fa = fa_ref[...]
    bidx = pl.program_id(0)
    s = _dot_bf(fb, fa)
    xxb = jnp.sum(fb * fb, axis=1, keepdims=True)
    ones8 = jnp.ones((8, CH), jnp.float32)
    xxa8 = lax.dot_general(ones8, fa * fa, (((1,), (1,)), ((), ())),
                           preferred_element_type=jnp.float32,
                           precision=lax.Precision.HIGHEST)
    pd = ((-xxb) - (jnp.float32(-2.0) * s)) - xxa8[0:1, :]
    iota = lax.broadcasted_iota(jnp.int32, (TOPK_RB, n), 1)
    neg = jnp.float32(-3.0e38)
    vals = [pd[c * TOPK_RB:(c + 1) * TOPK_RB] for c in range(TOPK_CHAINS)]
    cols = [[] for _ in range(TOPK_CHAINS)]
    for _ in range(KNN):
        for c in range(TOPK_CHAINS):
            m = jnp.max(vals[c], axis=1, keepdims=True)
            cand = jnp.where(vals[c] >= m, iota, n)
            sel = jnp.min(cand, axis=1, keepdims=True)
            cols[c].append(sel)
            vals[c] = jnp.where(iota == sel, neg, vals[c])
    idx = jnp.concatenate([jnp.concatenate(cs, axis=1) for cs in cols],
                          axis=0)
    idx_ref[0] = idx + bidx * n


def _topk(table2d, t, b, n):
    blk = TOPK_RB * TOPK_CHAINS
    nb = n // blk
    return pl.pallas_call(
        functools.partial(_topk_body, n),
        grid=(b, nb),
        in_specs=[
            pl.BlockSpec((blk, CH), lambda i, j: ((t * b + i) * nb + j, 0)),
            pl.BlockSpec((n, CH), lambda i, j: (t * b + i, 0)),
        ],
        out_specs=pl.BlockSpec((1, blk, KNN), lambda i, j: (i, j, 0)),
        out_shape=jax.ShapeDtypeStruct((b, n, KNN), jnp.int32),
    )(table2d, table2d)


def _sc_gather(table2d, cidx):
    info = plsc.get_sparse_core_info()
    nw = info.num_cores * info.num_subcores
    rows = cidx.shape[0]
    rw = rows // nw
    c = GATHER_CHUNK
    steps = rw // c
    mesh = plsc.VectorSubcoreMesh(core_axis_name="c", subcore_axis_name="s")

    @functools.partial(
        pl.kernel,
        out_type=jax.ShapeDtypeStruct((rows, CH), jnp.float32),
        mesh=mesh,
        scratch_types=[
            pltpu.VMEM((rw,), jnp.int32),
            pltpu.VMEM((2, c, CH), jnp.float32),
            pltpu.SemaphoreType.DMA,
            pltpu.SemaphoreType.DMA,
        ],
    )
    def gather_kernel(tab_hbm, idx_hbm, out_hbm, idx_v, rows_v, gsem, osem):
        wid = lax.axis_index("s") * info.num_cores + lax.axis_index("c")
        base = wid * rw
        pltpu.sync_copy(idx_hbm.at[pl.ds(base, rw)], idx_v)

        def start(i, slot):
            pltpu.async_copy(tab_hbm.at[idx_v.at[pl.ds(i * c, c)]],
                             rows_v.at[slot], gsem)

        def wait_gather(i, slot):
            pltpu.make_async_copy(tab_hbm.at[idx_v.at[pl.ds(i * c, c)]],
                                  rows_v.at[slot], gsem).wait()

        def start_out(i, slot):
            pltpu.async_copy(rows_v.at[slot],
                             out_hbm.at[pl.ds(base + i * c, c)], osem)

        def wait_out(i, slot):
            pltpu.make_async_copy(rows_v.at[slot],
                                  out_hbm.at[pl.ds(base + i * c, c)],
                                  osem).wait()

        start(0, 0)

        def body(i, _):
            slot = lax.rem(i, 2)
            nslot = lax.rem(i + 1, 2)
            start(i + 1, nslot)
            wait_gather(i, slot)
            start_out(i, slot)
            wait_out(i, slot)
            return 0

        lax.fori_loop(0, steps - 1, body, 0)
        last = steps - 1
        lslot = lax.rem(last, 2)
        wait_gather(last, lslot)
        start_out(last, lslot)
        wait_out(last, lslot)

    return gather_kernel(table2d, cidx)


def _attn_body(gk_ref, gv_ref, gp_ref, q_ref, x_ref,
               wpb_ref, spb_ref, bpb_ref, we1_ref, se1_ref, be1_ref,
               we2_ref, se2_ref, be2_ref, wm1_ref, sm1_ref, bm1_ref,
               wm2_ref, sm2_ref, bm2_ref, o_ref):
    qb = q_ref[0]
    xb = x_ref[0]

    def rep(a):
        return jnp.broadcast_to(a[:, None, :], (ATT_BLK, KNN, CH)).reshape(
            ATT_BLK * KNN, CH)

    mpos = gp_ref[...] - rep(xb)
    posb = _lrelu(_mmsb(mpos, wpb_ref[...], spb_ref[...], bpb_ref[...]))
    vec = gk_ref[...] - rep(qb) + posb
    val = gv_ref[...] + posb
    e = _lrelu(_mmsb(vec, we1_ref[...], se1_ref[...], be1_ref[...]))
    e = _lrelu(_mmsb(e, we2_ref[...], se2_ref[...], be2_ref[...]))
    e3 = e.reshape(ATT_BLK, KNN, G)
    m = jnp.max(e3, axis=1, keepdims=True)
    p = jnp.exp(e3 - m)
    w = p / jnp.sum(p, axis=1, keepdims=True)
    ci = lax.broadcasted_iota(jnp.int32, (CH, G), 0) // HEADS
    gi = lax.broadcasted_iota(jnp.int32, (CH, G), 1)
    expand_t = (ci == gi).astype(jnp.float32)
    wfull = _dot_bf(w.reshape(ATT_BLK * KNN, G), expand_t)
    valb = val.astype(jnp.bfloat16).astype(jnp.float32)
    attn = jnp.sum((wfull * valb).reshape(ATT_BLK, KNN, CH), axis=1)
    y = xb + attn
    h = jax.nn.silu(_mmsb(y, wm1_ref[...], sm1_ref[...], bm1_ref[...]))
    h = jax.nn.silu(_mmsb(h, wm2_ref[...], sm2_ref[...], bm2_ref[...]))
    o_ref[0] = y + h


def _attn(gath_kp, gath_v, q, x, wpb, spb2, bpb2, we1, se12, be12,
          we2, se22, be22, wm1, sm12, bm12, wm2, sm22, bm22):
    b, n, _ = x.shape
    nb = n // ATT_BLK

    def gspec(t):
        return pl.BlockSpec((ATT_BLK * KNN, CH),
                            lambda i, j, t=t: (t * b * nb + i * nb + j, 0))

    def cspec(shape):
        return pl.BlockSpec(shape, lambda i, j: (0, 0))

    return pl.pallas_call(
        _attn_body,
        grid=(b, nb),
        in_specs=[
            gspec(0), gspec(0), gspec(1),
            pl.BlockSpec((1, ATT_BLK, CH), lambda i, j: (i, j, 0)),
            pl.BlockSpec((1, ATT_BLK, CH), lambda i, j: (i, j, 0)),
            cspec((CH, CH)), cspec((1, CH)), cspec((1, CH)),
            cspec((G, CH)), cspec((1, G)), cspec((1, G)),
            cspec((G, G)), cspec((1, G)), cspec((1, G)),
            cspec((HID, CH)), cspec((1, HID)), cspec((1, HID)),
            cspec((CH, HID)), cspec((1, CH)), cspec((1, CH)),
        ],
        out_specs=pl.BlockSpec((1, ATT_BLK, CH), lambda i, j: (i, j, 0)),
        out_shape=jax.ShapeDtypeStruct((b, n, CH), jnp.float32),
    )(gath_kp, gath_v, gath_kp, q, x, wpb, spb2, bpb2, we1, se12, be12,
      we2, se22, be22, wm1, sm12, bm12, wm2, sm22, bm22)


def kernel(x, Wq, gq, bq, Wk, gk, bk, Wv, gv, bv, Wpb, gpb, bpb,
           We1, ge1, be1, We2, ge2, be2, Wm1, gm1, bm1, Wm2, gm2, bm2):
    b, n, _ = x.shape
    bn = b * n
    scale = 1.0 / jnp.sqrt(jnp.float32(1.0 + EPS))

    def sv(g):
        return (g * scale)[None, :]

    q, table = _qkv(x,
                    Wq, sv(gq), bq[None, :],
                    Wk, sv(gk), bk[None, :],
                    Wv, sv(gv), bv[None, :])
    table2d = table.reshape(3 * bn, CH)

    gk_idx = _topk(table2d, 0, b, n)
    cidx_kp = jnp.concatenate([
        gk_idx.reshape(-1),
        gk_idx.reshape(-1) + 2 * bn,
    ])
    gath_kp = _sc_gather(table2d, cidx_kp)

    gv_idx = _topk(table2d, 1, b, n)
    gath_v = _sc_gather(table2d, gv_idx.reshape(-1) + bn)

    return _attn(gath_kp, gath_v, q, x,
                 Wpb, sv(gpb), bpb[None, :],
                 We1, sv(ge1), be1[None, :],
                 We2, sv(ge2), be2[None, :],
                 Wm1, sv(gm1), bm1[None, :],
                 Wm2, sv(gm2), bm2[None, :])


# speedup vs baseline: 6.6771x; 1.0046x over previous
import functools

import jax
import jax.numpy as jnp
from jax import lax
from jax.experimental import pallas as pl
from jax.experimental.pallas import tpu as pltpu
from jax.experimental.pallas import tpu_sc as plsc

CH = 128
HEADS = 8
KNN = 32
G = CH // HEADS
HID = 512
EPS = 1e-5

QKV_BLK = 512
TOPK_RB = 256
TOPK_CHAINS = 4
ATT_BLK = 256
GATHER_CHUNK = 128


def _lrelu(x):
    return jnp.where(x >= 0, x, 0.2 * x)


def _dot_bf(a, w):
    return lax.dot_general(a.astype(jnp.bfloat16), w.astype(jnp.bfloat16),
                           (((1,), (1,)), ((), ())),
                           preferred_element_type=jnp.float32)


def _mmsb(a, w, s, bias):
    return _dot_bf(a, w) * s + bias


def _qkv_body(x_ref, wq_ref, sq_ref, bq_ref, wk_ref, sk_ref, bk_ref,
              wv_ref, sv_ref, bv_ref, q_ref, tab_ref):
    xb = x_ref[0]
    q_ref[0] = _lrelu(_mmsb(xb, wq_ref[...], sq_ref[...], bq_ref[...]))
    tab_ref[0] = _lrelu(_mmsb(xb, wk_ref[...], sk_ref[...], bk_ref[...]))
    tab_ref[1] = _lrelu(_mmsb(xb, wv_ref[...], sv_ref[...], bv_ref[...]))
    tab_ref[2] = xb


def _qkv(x, wq, sq2, bq2, wk, sk2, bk2, wv, sv2, bv2):
    b, n, _ = x.shape
    nb = n // QKV_BLK
    wspec = pl.BlockSpec((CH, CH), lambda i, j: (0, 0))
    bspec = pl.BlockSpec((1, CH), lambda i, j: (0, 0))
    return pl.pallas_call(
        _qkv_body,
        grid=(b, nb),
        in_specs=[
            pl.BlockSpec((1, QKV_BLK, CH), lambda i, j: (i, j, 0)),
            wspec, bspec, bspec, wspec, bspec, bspec, wspec, bspec, bspec,
        ],
        out_specs=[
            pl.BlockSpec((1, QKV_BLK, CH), lambda i, j: (i, j, 0)),
            pl.BlockSpec((3, QKV_BLK, CH), lambda i, j: (0, i * nb + j, 0)),
        ],
        out_shape=[
            jax.ShapeDtypeStruct((b, n, CH), jnp.float32),
            jax.ShapeDtypeStruct((3, b * n, CH), jnp.float32),
        ],
    )(x, wq, sq2, bq2, wk, sk2, bk2, wv, sv2, bv2)


def _topk_body(n, fb_ref, fa_ref, idx_ref):
    blk = TOPK_RB * TOPK_CHAINS
    fb = fb_ref[...]
    fa = fa_ref[...]
    bidx = pl.program_id(0)
    s = _dot_bf(fb, fa)
    xxb = jnp.sum(fb * fb, axis=1, keepdims=True)
    ones8 = jnp.ones((8, CH), jnp.float32)
    xxa8 = lax.dot_general(ones8, fa * fa, (((1,), (1,)), ((), ())),
                           preferred_element_type=jnp.float32,
                           precision=lax.Precision.HIGHEST)
    pd = ((-xxb) - (jnp.float32(-2.0) * s)) - xxa8[0:1, :]
    iota = lax.broadcasted_iota(jnp.int32, (TOPK_RB, n), 1)
    neg = jnp.float32(-3.0e38)
    vals = [pd[c * TOPK_RB:(c + 1) * TOPK_RB] for c in range(TOPK_CHAINS)]
    cols = [[] for _ in range(TOPK_CHAINS)]
    for _ in range(KNN):
        for c in range(TOPK_CHAINS):
            m = jnp.max(vals[c], axis=1, keepdims=True)
            cand = jnp.where(vals[c] >= m, iota, n)
            sel = jnp.min(cand, axis=1, keepdims=True)
            cols[c].append(sel)
            vals[c] = jnp.where(iota == sel, neg, vals[c])
    idx = jnp.concatenate([jnp.concatenate(cs, axis=1) for cs in cols],
                          axis=0)
    idx_ref[0] = idx + bidx * n


def _topk(table2d, t, b, n):
    blk = TOPK_RB * TOPK_CHAINS
    nb = n // blk
    return pl.pallas_call(
        functools.partial(_topk_body, n),
        grid=(b, nb),
        in_specs=[
            pl.BlockSpec((blk, CH), lambda i, j: ((t * b + i) * nb + j, 0)),
            pl.BlockSpec((n, CH), lambda i, j: (t * b + i, 0)),
        ],
        out_specs=pl.BlockSpec((1, blk, KNN), lambda i, j: (i, j, 0)),
        out_shape=jax.ShapeDtypeStruct((b, n, KNN), jnp.int32),
    )(table2d, table2d)


def _sc_gather(table2d, cidx):
    info = plsc.get_sparse_core_info()
    nw = info.num_cores * info.num_subcores
    rows = cidx.shape[0]
    rw = rows // nw
    c = GATHER_CHUNK
    steps = rw // c
    mesh = plsc.VectorSubcoreMesh(core_axis_name="c", subcore_axis_name="s")

    @functools.partial(
        pl.kernel,
        out_type=jax.ShapeDtypeStruct((rows, CH), jnp.float32),
        mesh=mesh,
        scratch_types=[
            pltpu.VMEM((rw,), jnp.int32),
            pltpu.VMEM((2, c, CH), jnp.float32),
            pltpu.SemaphoreType.DMA,
            pltpu.SemaphoreType.DMA,
        ],
    )
    def gather_kernel(tab_hbm, idx_hbm, out_hbm, idx_v, rows_v, gsem, osem):
        wid = lax.axis_index("s") * info.num_cores + lax.axis_index("c")
        base = wid * rw
        pltpu.sync_copy(idx_hbm.at[pl.ds(base, rw)], idx_v)

        def start(i, slot):
            pltpu.async_copy(tab_hbm.at[idx_v.at[pl.ds(i * c, c)]],
                             rows_v.at[slot], gsem)

        def wait_gather(i, slot):
            pltpu.make_async_copy(tab_hbm.at[idx_v.at[pl.ds(i * c, c)]],
                                  rows_v.at[slot], gsem).wait()

        def start_out(i, slot):
            pltpu.async_copy(rows_v.at[slot],
                             out_hbm.at[pl.ds(base + i * c, c)], osem)

        def wait_out(i, slot):
            pltpu.make_async_copy(rows_v.at[slot],
                                  out_hbm.at[pl.ds(base + i * c, c)],
                                  osem).wait()

        start(0, 0)

        def body(i, _):
            slot = lax.rem(i, 2)
            nslot = lax.rem(i + 1, 2)
            start(i + 1, nslot)
            wait_gather(i, slot)
            start_out(i, slot)
            wait_out(i, slot)
            return 0

        lax.fori_loop(0, steps - 1, body, 0)
        last = steps - 1
        lslot = lax.rem(last, 2)
        wait_gather(last, lslot)
        start_out(last, lslot)
        wait_out(last, lslot)

    return gather_kernel(table2d, cidx)


def _attn_body(gk_ref, gv_ref, gp_ref, q_ref, x_ref,
               wpb_ref, spb_ref, bpb_ref, we1_ref, se1_ref, be1_ref,
               we2_ref, se2_ref, be2_ref, wm1_ref, sm1_ref, bm1_ref,
               wm2_ref, sm2_ref, bm2_ref, o_ref):
    qb = q_ref[0]
    xb = x_ref[0]

    def rep(a):
        return jnp.broadcast_to(a[:, None, :], (ATT_BLK, KNN, CH)).reshape(
            ATT_BLK * KNN, CH)

    mpos = gp_ref[...] - rep(xb)
    posb = _lrelu(_mmsb(mpos, wpb_ref[...], spb_ref[...], bpb_ref[...]))
    vec = gk_ref[...] - rep(qb) + posb
    val = gv_ref[...] + posb
    e = _lrelu(_mmsb(vec, we1_ref[...], se1_ref[...], be1_ref[...]))
    e = _lrelu(_mmsb(e, we2_ref[...], se2_ref[...], be2_ref[...]))
    e3 = e.reshape(ATT_BLK, KNN, G)
    m = jnp.max(e3, axis=1, keepdims=True)
    p = jnp.exp(e3 - m)
    w = p / jnp.sum(p, axis=1, keepdims=True)
    ci = lax.broadcasted_iota(jnp.int32, (CH, G), 0) // HEADS
    gi = lax.broadcasted_iota(jnp.int32, (CH, G), 1)
    expand_t = (ci == gi).astype(jnp.float32)
    wfull = _dot_bf(w.reshape(ATT_BLK * KNN, G), expand_t)
    valb = val.astype(jnp.bfloat16).astype(jnp.float32)
    attn = jnp.sum((wfull * valb).reshape(ATT_BLK, KNN, CH), axis=1)
    y = xb + attn
    h = jax.nn.silu(_mmsb(y, wm1_ref[...], sm1_ref[...], bm1_ref[...]))
    h = jax.nn.silu(_mmsb(h, wm2_ref[...], sm2_ref[...], bm2_ref[...]))
    o_ref[0] = y + h


def _attn(gath_kp, gath_v, q, x, wpb, spb2, bpb2, we1, se12, be12,
          we2, se22, be22, wm1, sm12, bm12, wm2, sm22, bm22):
    b, n, _ = x.shape
    nb = n // ATT_BLK

    def gspec(t):
        return pl.BlockSpec((ATT_BLK * KNN, CH),
                            lambda i, j, t=t: (t * b * nb + i * nb + j, 0))

    def cspec(shape):
        return pl.BlockSpec(shape, lambda i, j: (0, 0))

    return pl.pallas_call(
        _attn_body,
        grid=(b, nb),
        in_specs=[
            gspec(0), gspec(0), gspec(1),
            pl.BlockSpec((1, ATT_BLK, CH), lambda i, j: (i, j, 0)),
            pl.BlockSpec((1, ATT_BLK, CH), lambda i, j: (i, j, 0)),
            cspec((CH, CH)), cspec((1, CH)), cspec((1, CH)),
            cspec((G, CH)), cspec((1, G)), cspec((1, G)),
            cspec((G, G)), cspec((1, G)), cspec((1, G)),
            cspec((HID, CH)), cspec((1, HID)), cspec((1, HID)),
            cspec((CH, HID)), cspec((1, CH)), cspec((1, CH)),
        ],
        out_specs=pl.BlockSpec((1, ATT_BLK, CH), lambda i, j: (i, j, 0)),
        out_shape=jax.ShapeDtypeStruct((b, n, CH), jnp.float32),
    )(gath_kp, gath_v, gath_kp, q, x, wpb, spb2, bpb2, we1, se12, be12,
      we2, se22, be22, wm1, sm12, bm12, wm2, sm22, bm22)


def kernel(x, Wq, gq, bq, Wk, gk, bk, Wv, gv, bv, Wpb, gpb, bpb,
           We1, ge1, be1, We2, ge2, be2, Wm1, gm1, bm1, Wm2, gm2, bm2):
    b, n, _ = x.shape
    bn = b * n
    scale = 1.0 / jnp.sqrt(jnp.float32(1.0 + EPS))

    def sv(g):
        return (g * scale)[None, :]

    q, table = _qkv(x,
                    Wq, sv(gq), bq[None, :],
                    Wk, sv(gk), bk[None, :],
                    Wv, sv(gv), bv[None, :])
    table2d = table.reshape(3 * bn, CH)

    gk_idx = _topk(table2d, 0, b, n)
    cidx_kp = jnp.concatenate([
        gk_idx.reshape(-1),
        gk_idx.reshape(-1) + 2 * bn,
    ])
    gath_kp = _sc_gather(table2d, cidx_kp)

    gv_idx = _topk(table2d, 1, b, n)
    gath_v = _sc_gather(table2d, gv_idx.reshape(-1) + bn)

    return _attn(gath_kp, gath_v, q, x,
                 Wpb, sv(gpb), bpb[None, :],
                 We1, sv(ge1), be1[None, :],
                 We2, sv(ge2), be2[None, :],
                 Wm1, sv(gm1), bm1[None, :],
                 Wm2, sv(gm2), bm2[None, :])
